# baseline (speedup 1.0000x reference)
import jax
import jax.numpy as jnp
from jax.experimental import pallas as pl
from jax.experimental.pallas import tpu as pltpu

_CHUNKS = (2, 2, 4, 8, 8, 8, 8, 8, 8, 4, 2, 2)
_MAXC = max(_CHUNKS)


def _body(p_hbm, t_hbm, o_hbm, tbuf, in0, in1, out0, out1, in_sems, out_sems, tsem):
    ins = (in0, in1)
    outs = (out0, out1)
    offs = []
    o = 0
    for c in _CHUNKS:
        offs.append(o)
        o += c
    nk = len(_CHUNKS)

    def in_copy(k):
        return pltpu.make_async_copy(
            p_hbm.at[pl.ds(offs[k], _CHUNKS[k])],
            ins[k % 2].at[pl.ds(0, _CHUNKS[k])],
            in_sems.at[k % 2],
        )

    def out_copy(k):
        return pltpu.make_async_copy(
            outs[k % 2].at[pl.ds(0, _CHUNKS[k])],
            o_hbm.at[pl.ds(offs[k], _CHUNKS[k])],
            out_sems.at[k % 2],
        )

    tcopy = pltpu.make_async_copy(t_hbm, tbuf, tsem)
    tcopy.start()
    in_copy(0).start()
    in_copy(1).start()
    tcopy.wait()

    for k in range(nk):
        in_copy(k).wait()
        if k >= 2:
            out_copy(k - 2).wait()
        i_buf = ins[k % 2]
        o_buf = outs[k % 2]

        def _one_batch(b, carry, i_buf=i_buf, o_buf=o_buf):
            o_buf[b] = i_buf[b] + tbuf[...]
            return carry

        jax.lax.fori_loop(0, _CHUNKS[k], _one_batch, 0)
        out_copy(k).start()
        if k + 2 < nk:
            in_copy(k + 2).start()

    out_copy(nk - 2).wait()
    out_copy(nk - 1).wait()


def kernel(patch, pos_table):
    batch, num_patches, proj_dim = patch.shape
    buf = pltpu.VMEM((_MAXC, num_patches, proj_dim), patch.dtype)
    return pl.pallas_call(
        _body,
        in_specs=[
            pl.BlockSpec(memory_space=pl.ANY),
            pl.BlockSpec(memory_space=pl.ANY),
        ],
        out_specs=pl.BlockSpec(memory_space=pl.ANY),
        out_shape=jax.ShapeDtypeStruct(patch.shape, patch.dtype),
        scratch_shapes=[
            pltpu.VMEM((num_patches, proj_dim), pos_table.dtype),
            buf,
            buf,
            buf,
            buf,
            pltpu.SemaphoreType.DMA((2,)),
            pltpu.SemaphoreType.DMA((2,)),
            pltpu.SemaphoreType.DMA,
        ],
    )(patch, pos_table)


# speedup vs baseline: 1.0069x; 1.0069x over previous
import jax
import jax.numpy as jnp
from jax.experimental import pallas as pl
from jax.experimental.pallas import tpu as pltpu

_CHUNKS = (4,) * 16
_MAXC = max(_CHUNKS)


def _body(p_hbm, t_hbm, o_hbm, tbuf, in0, in1, in2, out0, out1, out2, in_sems, out_sems, tsem):
    ins = (in0, in1, in2)
    outs = (out0, out1, out2)
    offs = []
    o = 0
    for c in _CHUNKS:
        offs.append(o)
        o += c
    nk = len(_CHUNKS)

    def in_copy(k):
        return pltpu.make_async_copy(
            p_hbm.at[pl.ds(offs[k], _CHUNKS[k])],
            ins[k % 3].at[pl.ds(0, _CHUNKS[k])],
            in_sems.at[k % 3],
        )

    def out_copy(k):
        return pltpu.make_async_copy(
            outs[k % 3].at[pl.ds(0, _CHUNKS[k])],
            o_hbm.at[pl.ds(offs[k], _CHUNKS[k])],
            out_sems.at[k % 3],
        )

    tcopy = pltpu.make_async_copy(t_hbm, tbuf, tsem)
    tcopy.start()
    in_copy(0).start()
    in_copy(1).start()
    in_copy(2).start()
    tcopy.wait()

    for k in range(nk):
        in_copy(k).wait()
        if k >= 3:
            out_copy(k - 3).wait()
        i_buf = ins[k % 3]
        o_buf = outs[k % 3]

        def _one_batch(b, carry, i_buf=i_buf, o_buf=o_buf):
            o_buf[b] = i_buf[b] + tbuf[...]
            return carry

        jax.lax.fori_loop(0, _CHUNKS[k], _one_batch, 0)
        out_copy(k).start()
        if k + 3 < nk:
            in_copy(k + 3).start()

    out_copy(nk - 3).wait()
    out_copy(nk - 2).wait()
    out_copy(nk - 1).wait()


def kernel(patch, pos_table):
    batch, num_patches, proj_dim = patch.shape
    buf = pltpu.VMEM((_MAXC, num_patches, proj_dim), patch.dtype)
    return pl.pallas_call(
        _body,
        in_specs=[
            pl.BlockSpec(memory_space=pl.ANY),
            pl.BlockSpec(memory_space=pl.ANY),
        ],
        out_specs=pl.BlockSpec(memory_space=pl.ANY),
        out_shape=jax.ShapeDtypeStruct(patch.shape, patch.dtype),
        scratch_shapes=[
            pltpu.VMEM((num_patches, proj_dim), pos_table.dtype),
            buf,
            buf,
            buf,
            buf,
            buf,
            buf,
            pltpu.SemaphoreType.DMA((3,)),
            pltpu.SemaphoreType.DMA((3,)),
            pltpu.SemaphoreType.DMA,
        ],
    )(patch, pos_table)
